# Initial kernel scaffold; baseline (speedup 1.0000x reference)
import jax
import jax.numpy as jnp
from jax.experimental import pallas as pl


def kernel(pred_psi_val, psi_val, event_id, use_BCE_loss_only):
    raise NotImplementedError("write your pallas kernel here")


# speedup vs baseline: 1.2646x; 1.2646x over previous
import functools

import jax
import jax.numpy as jnp
from jax import lax
from jax.experimental import pallas as pl
from jax.experimental.pallas import tpu as pltpu
from jax.experimental.pallas import tpu_sc as plsc

B = 4096
N_EVENTS = 512
DPSI_THRESHOLD = 0.05
MSE_WEIGHT = 10.0
EPS = 1e-7

LANES = 16
NWORKERS = 32
ROWS_PER = B // NWORKERS
GROUPS = ROWS_PER // LANES
NCHUNK = B // LANES


def _prep_body(pred_ref, psi_ref, u_ref, bce_ref):
    x = pred_ref[...]
    y = psi_ref[...]
    bce_ref[0, 0] = jnp.sum(
        jnp.maximum(x, 0.0) - x * y + jnp.log1p(jnp.exp(-jnp.abs(x)))
    )
    p = jnp.clip(y, EPS, 1.0 - EPS)
    u_ref[...] = x - (jnp.log(p) - jnp.log1p(-p))


_prep = pl.pallas_call(
    _prep_body,
    out_shape=[
        jax.ShapeDtypeStruct((B // 128, 128), jnp.float32),
        jax.ShapeDtypeStruct((1, 1), jnp.float32),
    ],
    out_specs=[
        pl.BlockSpec(memory_space=pltpu.VMEM),
        pl.BlockSpec(memory_space=pltpu.SMEM),
    ],
)


def _pairs_body(u_hbm, psi_hbm, ev_hbm, out_sq, out_ct, u_v, psi_v, ev_v, osq_v, oct_v):
    wid = lax.axis_index("s") * 2 + lax.axis_index("c")
    pltpu.sync_copy(u_hbm, u_v)
    pltpu.sync_copy(psi_hbm, psi_v)
    pltpu.sync_copy(ev_hbm, ev_v)

    base = wid * ROWS_PER
    lane = lax.iota(jnp.int32, LANES)
    zero = jnp.zeros((LANES,), jnp.float32)

    def group_body(g, carry):
        a_sq, a_ct = carry
        rbase = base + g * LANES
        u_r = u_v[pl.ds(rbase, LANES)]
        psi_r = psi_v[pl.ds(rbase, LANES)]
        ev_r = ev_v[pl.ds(rbase, LANES)]
        ev_lo = ev_r[0]
        ev_hi = ev_r[LANES - 1]

        c_lo = jnp.int32(0)
        c_hi = jnp.int32(0)
        for k in (256, 128, 64, 32, 16, 8, 4, 2, 1):
            nlo = c_lo + k
            cmax = ev_v[pl.ds((jnp.minimum(nlo, NCHUNK) - 1) * LANES, LANES)][LANES - 1]
            c_lo = jnp.where((nlo <= NCHUNK) & (cmax < ev_lo), nlo, c_lo)
            nhi = c_hi + k
            cmin = ev_v[pl.ds((jnp.minimum(nhi, NCHUNK) - 1) * LANES, LANES)][0]
            c_hi = jnp.where((nhi <= NCHUNK) & (cmin <= ev_hi), nhi, c_hi)

        def chunk_body(c, acc):
            b_sq, b_ct = acc
            cb = c * LANES
            u_c = u_v[pl.ds(cb, LANES)]
            psi_c = psi_v[pl.ds(cb, LANES)]
            ev_c = ev_v[pl.ds(cb, LANES)]
            for s in range(LANES):
                idx = (lane + s) & (LANES - 1)
                u_x = u_c.at[idx].get(mode="promise_in_bounds")
                psi_x = psi_c.at[idx].get(mode="promise_in_bounds")
                ev_x = ev_c.at[idx].get(mode="promise_in_bounds")
                m = (ev_x == ev_r) & (jnp.abs(psi_x - psi_r) >= DPSI_THRESHOLD)
                d = u_x - u_r
                b_sq = b_sq + jnp.where(m, d * d, 0.0)
                b_ct = b_ct + jnp.where(m, 1.0, 0.0)
            return (b_sq, b_ct)

        a_sq, a_ct = lax.fori_loop(c_lo, c_hi, chunk_body, (a_sq, a_ct))
        return (a_sq, a_ct)

    acc_sq, acc_ct = lax.fori_loop(0, GROUPS, group_body, (zero, zero))
    osq_v[...] = acc_sq
    oct_v[...] = acc_ct
    pltpu.sync_copy(osq_v, out_sq.at[wid])
    pltpu.sync_copy(oct_v, out_ct.at[wid])


_pairs = functools.partial(
    pl.kernel,
    mesh=plsc.VectorSubcoreMesh(core_axis_name="c", subcore_axis_name="s"),
    out_type=[
        jax.ShapeDtypeStruct((NWORKERS, LANES), jnp.float32),
        jax.ShapeDtypeStruct((NWORKERS, LANES), jnp.float32),
    ],
    scratch_types=[
        pltpu.VMEM((B,), jnp.float32),
        pltpu.VMEM((B,), jnp.float32),
        pltpu.VMEM((B,), jnp.int32),
        pltpu.VMEM((LANES,), jnp.float32),
        pltpu.VMEM((LANES,), jnp.float32),
    ],
)(_pairs_body)


def kernel(pred_psi_val, psi_val, event_id, use_BCE_loss_only):
    u2d, bce_sum = _prep(
        pred_psi_val.reshape(B // 128, 128), psi_val.reshape(B // 128, 128)
    )
    u = u2d.reshape(B)
    part_sq, part_ct = _pairs(u, psi_val, event_id.astype(jnp.int32))
    bce = bce_sum[0, 0] / B
    cnt = jnp.sum(part_ct)
    pairwise_mse = jnp.sum(part_sq) / jnp.maximum(cnt, 1.0)
    full_loss = bce + jnp.where(cnt > 0, pairwise_mse * MSE_WEIGHT, 0.0)
    return jnp.where(use_BCE_loss_only != 0, bce, full_loss)
